# baseline (speedup 1.0000x reference)
import functools

import jax
import jax.numpy as jnp
from jax import lax
from jax.experimental import pallas as pl
from jax.experimental.pallas import tpu as pltpu
from jax.experimental.pallas import tpu_sc as plsc

N = 10000
G = 64
D = 128
NCORES = 2
NSUB = 16
NW = NCORES * NSUB
CH = 128
NCH = 80
HALF = NCH // 2
EPW = CH * NCH
EP = NW * EPW
NROW = 10240
DEGW = 16
ZPT = NROW // NSUB

_MESH = plsc.VectorSubcoreMesh(core_axis_name="c", subcore_axis_name="s")


def _deg_body(dst_hbm, out_hbm, dstv, buf, acc):
    c = lax.axis_index("c")
    s = lax.axis_index("s")
    wid = c * NSUB + s
    pltpu.sync_copy(dst_hbm.at[pl.ds(wid * NCH, NCH)], dstv)

    def fill(val):
        def row(i, _):
            buf[i, :] = jnp.full((DEGW,), val, jnp.float32)
            return ()
        lax.fori_loop(0, CH, row, ())

    fill(0.0)
    for j in range(ZPT // CH):
        pltpu.sync_copy(buf, acc.at[pl.ds(s * ZPT + j * CH, CH)])
    fill(1.0)
    plsc.subcore_barrier()

    def body(k, _):
        pltpu.sync_copy(buf, acc.at[dstv.at[k]], add=True)
        return ()
    lax.fori_loop(0, NCH, body, ())

    plsc.subcore_barrier()
    pltpu.sync_copy(acc.at[pl.ds(s * ZPT, ZPT)],
                    out_hbm.at[pl.ds(c * NROW + s * ZPT, ZPT)])


_deg_call = functools.partial(
    pl.kernel,
    out_type=jax.ShapeDtypeStruct((NCORES * NROW, DEGW), jnp.float32),
    mesh=_MESH,
    scratch_types=[
        pltpu.VMEM((NCH, CH), jnp.int32),
        pltpu.VMEM((CH, DEGW), jnp.float32),
        pltpu.VMEM_SHARED((NROW, DEGW), jnp.float32),
    ],
)(_deg_body)


def _agg_body(hs_hbm, src_hbm, dst_hbm, out_hbm,
              srcv, dstv, bufa, bufb, acc, sema, semb):
    c = lax.axis_index("c")
    s = lax.axis_index("s")
    wid = c * NSUB + s

    def zrow(i, _):
        for j in range(D // 16):
            bufa[i, pl.ds(j * 16, 16)] = jnp.zeros((16,), jnp.float32)
        return ()
    lax.fori_loop(0, CH, zrow, ())
    for j in range(ZPT // CH):
        pltpu.sync_copy(bufa, acc.at[pl.ds(s * ZPT + j * CH, CH)])
    plsc.subcore_barrier()

    for p in range(2):
        base = wid * NCH + p * HALF
        pltpu.sync_copy(src_hbm.at[pl.ds(base, HALF)], srcv)
        pltpu.sync_copy(dst_hbm.at[pl.ds(base, HALF)], dstv)

        pltpu.async_copy(hs_hbm.at[srcv.at[0]], bufa, sema)

        def body(i, _):
            k = 2 * i
            pltpu.async_copy(hs_hbm.at[srcv.at[k + 1]], bufb, semb)
            pltpu.make_async_copy(hs_hbm.at[srcv.at[0]], bufa, sema).wait()
            pltpu.sync_copy(bufa, acc.at[dstv.at[k]], add=True)

            @pl.when(k + 2 < HALF)
            def _():
                pltpu.async_copy(hs_hbm.at[srcv.at[k + 2]], bufa, sema)

            pltpu.make_async_copy(hs_hbm.at[srcv.at[0]], bufb, semb).wait()
            pltpu.sync_copy(bufb, acc.at[dstv.at[k + 1]], add=True)
            return ()
        lax.fori_loop(0, HALF // 2, body, ())

    plsc.subcore_barrier()
    pltpu.sync_copy(acc.at[pl.ds(s * ZPT, ZPT)],
                    out_hbm.at[pl.ds(c * NROW + s * ZPT, ZPT)])


_agg = functools.partial(
    pl.kernel,
    out_type=jax.ShapeDtypeStruct((NCORES * NROW, D), jnp.float32),
    mesh=_MESH,
    scratch_types=[
        pltpu.VMEM((HALF, CH), jnp.int32),
        pltpu.VMEM((HALF, CH), jnp.int32),
        pltpu.VMEM((CH, D), jnp.float32),
        pltpu.VMEM((CH, D), jnp.float32),
        pltpu.VMEM_SHARED((NROW, D), jnp.float32),
        pltpu.SemaphoreType.DMA,
        pltpu.SemaphoreType.DMA,
    ],
)(_agg_body)


_RB = 1000
_NBLK = N // _RB


def _dinv(degp_ref):
    d = degp_ref[0, :, 0:1] + degp_ref[1, :, 0:1] + 1.0
    return lax.rsqrt(d)


def _tc_first_body(x_ref, w_ref, degp_ref, out_ref):
    h = jnp.dot(x_ref[...], w_ref[...], preferred_element_type=jnp.float32)
    out_ref[...] = h * _dinv(degp_ref)


def _tc_first(x, w, degp):
    return pl.pallas_call(
        _tc_first_body,
        grid=(_NBLK,),
        in_specs=[
            pl.BlockSpec((_RB, 128), lambda i: (i, 0)),
            pl.BlockSpec((128, 128), lambda i: (0, 0)),
            pl.BlockSpec((2, _RB, DEGW), lambda i: (0, i, 0)),
        ],
        out_specs=pl.BlockSpec((_RB, 128), lambda i: (i, 0)),
        out_shape=jax.ShapeDtypeStruct((N, 128), jnp.float32),
    )(x, w, degp)


def _tc_mid_body(raw_ref, hs_ref, degp_ref, b_ref, w_ref, out_ref):
    dinv = _dinv(degp_ref)
    agg = (raw_ref[0] + raw_ref[1] + hs_ref[...]) * dinv
    act = jnp.maximum(agg + b_ref[...], 0.0)
    h = jnp.dot(act, w_ref[...], preferred_element_type=jnp.float32)
    out_ref[...] = h * dinv


def _tc_mid(raw, hs, degp, b, w):
    return pl.pallas_call(
        _tc_mid_body,
        grid=(_NBLK,),
        in_specs=[
            pl.BlockSpec((2, _RB, 128), lambda i: (0, i, 0)),
            pl.BlockSpec((_RB, 128), lambda i: (i, 0)),
            pl.BlockSpec((2, _RB, DEGW), lambda i: (0, i, 0)),
            pl.BlockSpec((1, 128), lambda i: (0, 0)),
            pl.BlockSpec((128, 128), lambda i: (0, 0)),
        ],
        out_specs=pl.BlockSpec((_RB, 128), lambda i: (i, 0)),
        out_shape=jax.ShapeDtypeStruct((N, 128), jnp.float32),
    )(raw, hs, degp, b, w)


_DF = 64


def _tc_final_body(raw_ref, hs_ref, degp_ref, b_ref, batch_ref,
                   hw1_ref, hb1_ref, hw2_ref, hb2_ref, out_ref,
                   sums_ref, counts_ref):
    i = pl.program_id(0)
    dinv = _dinv(degp_ref)
    act = jnp.maximum(
        (raw_ref[0] + raw_ref[1] + hs_ref[...]) * dinv + b_ref[...], 0.0)

    gids = lax.broadcasted_iota(jnp.int32, (1, G), 1)
    oh = jnp.where(batch_ref[...] == gids, 1.0, 0.0)

    psum = lax.dot_general(oh, act, (((0,), (0,)), ((), ())),
                           preferred_element_type=jnp.float32)
    ones = jnp.ones((_RB, 1), jnp.float32)
    pcnt = lax.dot_general(oh, ones, (((0,), (0,)), ((), ())),
                           preferred_element_type=jnp.float32)

    @pl.when(i == 0)
    def _():
        sums_ref[...] = jnp.zeros_like(sums_ref)
        counts_ref[...] = jnp.zeros_like(counts_ref)

    sums_ref[...] += psum
    counts_ref[...] += pcnt

    @pl.when(i == _NBLK - 1)
    def _():
        g = sums_ref[...] / jnp.maximum(counts_ref[...], 1.0)
        for h in range(5):
            hid = jnp.dot(g, hw1_ref[h],
                          preferred_element_type=jnp.float32)
            hid = jnp.maximum(hid + hb1_ref[h], 0.0)
            row = lax.dot_general(hw2_ref[h], hid, (((0,), (1,)), ((), ())),
                                  preferred_element_type=jnp.float32)
            out_ref[pl.ds(h, 1), :] = row + hb2_ref[h, 0]


def _tc_final(raw, hs, degp, b, batch2d, hw1, hb1, hw2, hb2):
    return pl.pallas_call(
        _tc_final_body,
        grid=(_NBLK,),
        in_specs=[
            pl.BlockSpec((2, _RB, 128), lambda i: (0, i, 0)),
            pl.BlockSpec((_RB, 128), lambda i: (i, 0)),
            pl.BlockSpec((2, _RB, DEGW), lambda i: (0, i, 0)),
            pl.BlockSpec((1, 128), lambda i: (0, 0)),
            pl.BlockSpec((_RB, 1), lambda i: (i, 0)),
            pl.BlockSpec((5, 128, 32), lambda i: (0, 0, 0)),
            pl.BlockSpec((5, 32), lambda i: (0, 0)),
            pl.BlockSpec((5, 32, 1), lambda i: (0, 0, 0)),
            pl.BlockSpec((5, 1), lambda i: (0, 0)),
        ],
        out_specs=pl.BlockSpec((5, G), lambda i: (0, 0)),
        out_shape=jax.ShapeDtypeStruct((5, G), jnp.float32),
        scratch_shapes=[
            pltpu.VMEM((G, 128), jnp.float32),
            pltpu.VMEM((G, 1), jnp.float32),
        ],
    )(raw, hs, degp, b, batch2d, hw1, hb1, hw2, hb2)


def kernel(x, edge_index, batch, W1, b1, W2, b2, W3, b3, hW1, hb1, hW2, hb2):
    e = edge_index.shape[1]
    src = jnp.concatenate(
        [edge_index[0], jnp.zeros((EP - e,), jnp.int32)]).reshape(NW * NCH, CH)
    pad_dst = jnp.arange(EP - e, dtype=jnp.int32) % (NROW - N) + N
    dst = jnp.concatenate([edge_index[1], pad_dst]).reshape(NW * NCH, CH)

    degp = _deg_call(dst).reshape(NCORES, NROW, DEGW)

    hs1 = _tc_first(x, W1, degp)
    raw1 = _agg(hs1, src, dst).reshape(NCORES, NROW, D)
    hs2 = _tc_mid(raw1, hs1, degp, b1.reshape(1, -1), W2)
    raw2 = _agg(hs2, src, dst).reshape(NCORES, NROW, D)
    w3pad = jnp.concatenate(
        [W3, jnp.zeros((128, 128 - _DF), jnp.float32)], axis=1)
    hs3 = _tc_mid(raw2, hs2, degp, b2.reshape(1, -1), w3pad)
    raw3 = _agg(hs3, src, dst).reshape(NCORES, NROW, D)

    b3pad = jnp.concatenate(
        [b3, jnp.zeros((128 - _DF,), jnp.float32)]).reshape(1, -1)
    hw1pad = jnp.concatenate(
        [hW1, jnp.zeros((5, 128 - _DF, 32), jnp.float32)], axis=1)
    out = _tc_final(raw3, hs3, degp, b3pad,
                    batch.reshape(-1, 1), hw1pad, hb1, hW2, hb2)

    tasks = ['solubility', 'permeability', 'logp', 'cyp3a4', 'herg']
    return {t: out[i] for i, t in enumerate(tasks)}


# speedup vs baseline: 3.3997x; 3.3997x over previous
import functools

import jax
import jax.numpy as jnp
from jax import lax
from jax.experimental import pallas as pl
from jax.experimental.pallas import tpu as pltpu
from jax.experimental.pallas import tpu_sc as plsc

N = 10000
G = 64
D = 128
NCORES = 2
NSUB = 16
NW = NCORES * NSUB
CH = 128
NCH = 80
HALF = NCH // 2
EPW = CH * NCH
EP = NW * EPW
NROW = 10240
DEGW = 16
ZPT = NROW // NSUB

_MESH = plsc.VectorSubcoreMesh(core_axis_name="c", subcore_axis_name="s")


def _deg_body(dst_hbm, out_hbm, dstv, buf, acc):
    c = lax.axis_index("c")
    s = lax.axis_index("s")
    wid = c * NSUB + s
    pltpu.sync_copy(dst_hbm.at[pl.ds(wid * NCH, NCH)], dstv)

    def fill(val):
        def row(i, _):
            buf[i, :] = jnp.full((DEGW,), val, jnp.float32)
            return ()
        lax.fori_loop(0, CH, row, ())

    fill(0.0)
    for j in range(ZPT // CH):
        pltpu.sync_copy(buf, acc.at[pl.ds(s * ZPT + j * CH, CH)])
    fill(1.0)
    plsc.subcore_barrier()

    def body(k, _):
        pltpu.sync_copy(buf, acc.at[dstv.at[k]], add=True)
        return ()
    lax.fori_loop(0, NCH, body, ())

    plsc.subcore_barrier()
    pltpu.sync_copy(acc.at[pl.ds(s * ZPT, ZPT)],
                    out_hbm.at[pl.ds(c * NROW + s * ZPT, ZPT)])


_deg_call = functools.partial(
    pl.kernel,
    out_type=jax.ShapeDtypeStruct((NCORES * NROW, DEGW), jnp.float32),
    mesh=_MESH,
    scratch_types=[
        pltpu.VMEM((NCH, CH), jnp.int32),
        pltpu.VMEM((CH, DEGW), jnp.float32),
        pltpu.VMEM_SHARED((NROW, DEGW), jnp.float32),
    ],
)(_deg_body)


def _agg_body(hs_hbm, src_hbm, dst_hbm, out_hbm,
              srcv, dstv, bufa, bufb, acc, sema, semb):
    c = lax.axis_index("c")
    s = lax.axis_index("s")
    wid = c * NSUB + s

    def zrow(i, _):
        for j in range(D // 16):
            bufa[i, pl.ds(j * 16, 16)] = jnp.zeros((16,), jnp.float32)
        return ()
    lax.fori_loop(0, CH, zrow, ())
    for j in range(ZPT // CH):
        pltpu.sync_copy(bufa, acc.at[pl.ds(s * ZPT + j * CH, CH)])
    plsc.subcore_barrier()

    for p in range(2):
        base = wid * NCH + p * HALF
        pltpu.sync_copy(src_hbm.at[pl.ds(base, HALF)], srcv)
        pltpu.sync_copy(dst_hbm.at[pl.ds(base, HALF)], dstv)

        pltpu.async_copy(hs_hbm.at[srcv.at[0]], bufa, sema)

        def body(i, _):
            k = 2 * i
            pltpu.async_copy(hs_hbm.at[srcv.at[k + 1]], bufb, semb)
            pltpu.make_async_copy(hs_hbm.at[srcv.at[0]], bufa, sema).wait()
            pltpu.sync_copy(bufa, acc.at[dstv.at[k]], add=True)

            @pl.when(k + 2 < HALF)
            def _():
                pltpu.async_copy(hs_hbm.at[srcv.at[k + 2]], bufa, sema)

            pltpu.make_async_copy(hs_hbm.at[srcv.at[0]], bufb, semb).wait()
            pltpu.sync_copy(bufb, acc.at[dstv.at[k + 1]], add=True)
            return ()
        lax.fori_loop(0, HALF // 2, body, ())

    plsc.subcore_barrier()
    pltpu.sync_copy(acc.at[pl.ds(s * ZPT, ZPT)],
                    out_hbm.at[pl.ds(c * NROW + s * ZPT, ZPT)])


_agg = functools.partial(
    pl.kernel,
    out_type=jax.ShapeDtypeStruct((NCORES * NROW, D), jnp.float32),
    mesh=_MESH,
    scratch_types=[
        pltpu.VMEM((HALF, CH), jnp.int32),
        pltpu.VMEM((HALF, CH), jnp.int32),
        pltpu.VMEM((CH, D), jnp.float32),
        pltpu.VMEM((CH, D), jnp.float32),
        pltpu.VMEM_SHARED((NROW, D), jnp.float32),
        pltpu.SemaphoreType.DMA,
        pltpu.SemaphoreType.DMA,
    ],
)(_agg_body)


_RB = 1000
_NBLK = N // _RB


def _dinv(degp_ref):
    d = degp_ref[0, :, 0:1] + degp_ref[1, :, 0:1] + 1.0
    return lax.rsqrt(d)


def _tc_first_body(x_ref, w_ref, degp_ref, out_ref):
    h = jnp.dot(x_ref[...], w_ref[...], preferred_element_type=jnp.float32)
    out_ref[...] = h * _dinv(degp_ref)


def _tc_first(x, w, degp):
    return pl.pallas_call(
        _tc_first_body,
        grid=(_NBLK,),
        in_specs=[
            pl.BlockSpec((_RB, 128), lambda i: (i, 0)),
            pl.BlockSpec((128, 128), lambda i: (0, 0)),
            pl.BlockSpec((2, _RB, DEGW), lambda i: (0, i, 0)),
        ],
        out_specs=pl.BlockSpec((_RB, 128), lambda i: (i, 0)),
        out_shape=jax.ShapeDtypeStruct((N, 128), jnp.float32),
    )(x, w, degp)


def _tc_mid_body(raw_ref, hs_ref, degp_ref, b_ref, w_ref, out_ref):
    dinv = _dinv(degp_ref)
    agg = (raw_ref[0] + raw_ref[1] + hs_ref[...]) * dinv
    act = jnp.maximum(agg + b_ref[...], 0.0)
    h = jnp.dot(act, w_ref[...], preferred_element_type=jnp.float32)
    out_ref[...] = h * dinv


def _tc_mid(raw, hs, degp, b, w):
    return pl.pallas_call(
        _tc_mid_body,
        grid=(_NBLK,),
        in_specs=[
            pl.BlockSpec((2, _RB, 128), lambda i: (0, i, 0)),
            pl.BlockSpec((_RB, 128), lambda i: (i, 0)),
            pl.BlockSpec((2, _RB, DEGW), lambda i: (0, i, 0)),
            pl.BlockSpec((1, 128), lambda i: (0, 0)),
            pl.BlockSpec((128, 128), lambda i: (0, 0)),
        ],
        out_specs=pl.BlockSpec((_RB, 128), lambda i: (i, 0)),
        out_shape=jax.ShapeDtypeStruct((N, 128), jnp.float32),
    )(raw, hs, degp, b, w)


_DF = 64


def _tc_final_body(raw_ref, hs_ref, degp_ref, b_ref, batch_ref,
                   hw1_ref, hb1_ref, hw2_ref, hb2_ref, out_ref,
                   sums_ref, counts_ref):
    i = pl.program_id(0)
    dinv = _dinv(degp_ref)
    act = jnp.maximum(
        (raw_ref[0] + raw_ref[1] + hs_ref[...]) * dinv + b_ref[...], 0.0)

    gids = lax.broadcasted_iota(jnp.int32, (1, G), 1)
    oh = jnp.where(batch_ref[...] == gids, 1.0, 0.0)

    psum = lax.dot_general(oh, act, (((0,), (0,)), ((), ())),
                           preferred_element_type=jnp.float32)
    ones = jnp.ones((_RB, 1), jnp.float32)
    pcnt = lax.dot_general(oh, ones, (((0,), (0,)), ((), ())),
                           preferred_element_type=jnp.float32)

    @pl.when(i == 0)
    def _():
        sums_ref[...] = jnp.zeros_like(sums_ref)
        counts_ref[...] = jnp.zeros_like(counts_ref)

    sums_ref[...] += psum
    counts_ref[...] += pcnt

    @pl.when(i == _NBLK - 1)
    def _():
        g = sums_ref[...] / jnp.maximum(counts_ref[...], 1.0)
        for h in range(5):
            hid = jnp.dot(g, hw1_ref[h],
                          preferred_element_type=jnp.float32)
            hid = jnp.maximum(hid + hb1_ref[h], 0.0)
            row = lax.dot_general(hw2_ref[h], hid, (((0,), (1,)), ((), ())),
                                  preferred_element_type=jnp.float32)
            out_ref[pl.ds(h, 1), :] = row + hb2_ref[h, 0]


def _tc_final(raw, hs, degp, b, batch2d, hw1, hb1, hw2, hb2):
    return pl.pallas_call(
        _tc_final_body,
        grid=(_NBLK,),
        in_specs=[
            pl.BlockSpec((2, _RB, 128), lambda i: (0, i, 0)),
            pl.BlockSpec((_RB, 128), lambda i: (i, 0)),
            pl.BlockSpec((2, _RB, DEGW), lambda i: (0, i, 0)),
            pl.BlockSpec((1, 128), lambda i: (0, 0)),
            pl.BlockSpec((_RB, 1), lambda i: (i, 0)),
            pl.BlockSpec((5, 128, 32), lambda i: (0, 0, 0)),
            pl.BlockSpec((5, 32), lambda i: (0, 0)),
            pl.BlockSpec((5, 32, 1), lambda i: (0, 0, 0)),
            pl.BlockSpec((5, 1), lambda i: (0, 0)),
        ],
        out_specs=pl.BlockSpec((5, G), lambda i: (0, 0)),
        out_shape=jax.ShapeDtypeStruct((5, G), jnp.float32),
        scratch_shapes=[
            pltpu.VMEM((G, 128), jnp.float32),
            pltpu.VMEM((G, 1), jnp.float32),
        ],
    )(raw, hs, degp, b, batch2d, hw1, hb1, hw2, hb2)


def kernel(x, edge_index, batch, W1, b1, W2, b2, W3, b3, hW1, hb1, hW2, hb2):
    e = edge_index.shape[1]
    pad_src = jnp.arange(EP - e, dtype=jnp.int32) % N
    src = jnp.concatenate([edge_index[0], pad_src]).reshape(NW * NCH, CH)
    pad_dst = jnp.arange(EP - e, dtype=jnp.int32) % (NROW - N) + N
    dst = jnp.concatenate([edge_index[1], pad_dst]).reshape(NW * NCH, CH)

    degp = _deg_call(dst).reshape(NCORES, NROW, DEGW)

    hs1 = _tc_first(x, W1, degp)
    raw1 = _agg(hs1, src, dst).reshape(NCORES, NROW, D)
    hs2 = _tc_mid(raw1, hs1, degp, b1.reshape(1, -1), W2)
    raw2 = _agg(hs2, src, dst).reshape(NCORES, NROW, D)
    w3pad = jnp.concatenate(
        [W3, jnp.zeros((128, 128 - _DF), jnp.float32)], axis=1)
    hs3 = _tc_mid(raw2, hs2, degp, b2.reshape(1, -1), w3pad)
    raw3 = _agg(hs3, src, dst).reshape(NCORES, NROW, D)

    b3pad = jnp.concatenate(
        [b3, jnp.zeros((128 - _DF,), jnp.float32)]).reshape(1, -1)
    hw1pad = jnp.concatenate(
        [hW1, jnp.zeros((5, 128 - _DF, 32), jnp.float32)], axis=1)
    out = _tc_final(raw3, hs3, degp, b3pad,
                    batch.reshape(-1, 1), hw1pad, hb1, hW2, hb2)

    tasks = ['solubility', 'permeability', 'logp', 'cyp3a4', 'herg']
    return {t: out[i] for i, t in enumerate(tasks)}
